# Initial kernel scaffold; baseline (speedup 1.0000x reference)
import functools

import jax
import jax.numpy as jnp
from jax import lax
from jax.experimental import pallas as pl
from jax.experimental.pallas import tpu as pltpu
from jax.experimental.pallas import tpu_sc as plsc

_E = 8
_K = 2
_H = 2048
_I = 1408
_N = 4096
_NK = _N * _K
_BM = 256
_NPAD = _NK + _E * _BM
_NB = _NPAD // _BM
_BN1 = 704
_NN1 = _I // _BN1
_BN2 = 1024
_NN2 = _H // _BN2

_NW = 32
_GR = _NPAD // _NW
_GCH = 32
_TPW = _N // _NW
_CT = 16


def _sc_mesh():
    return plsc.VectorSubcoreMesh(core_axis_name="c", subcore_axis_name="s")


def _sc_gather(hidden_states, src_tok):
    @functools.partial(
        pl.kernel,
        out_type=jax.ShapeDtypeStruct((_NPAD, _H), jnp.float32),
        mesh=_sc_mesh(),
        scratch_types=[
            pltpu.VMEM((_GR,), jnp.int32),
            pltpu.VMEM((_GCH, _H), jnp.float32),
            pltpu.SemaphoreType.DMA,
        ],
    )
    def gather_kernel(x_hbm, idx_hbm, out_hbm, idx_v, buf, sem):
        wid = lax.axis_index("s") * 2 + lax.axis_index("c")
        base = wid * _GR
        pltpu.sync_copy(idx_hbm.at[pl.ds(base, _GR)], idx_v)

        def chunk(c, carry):
            cbase = c * _GCH
            pltpu.async_copy(
                x_hbm.at[idx_v.at[pl.ds(cbase, _GCH)]], buf, sem
            ).wait()
            pltpu.sync_copy(buf, out_hbm.at[pl.ds(base + cbase, _GCH)])
            return carry

        lax.fori_loop(0, _GR // _GCH, chunk, 0)

    return gather_kernel(hidden_states, src_tok)


def _sc_combine(y_pad, inv):
    @functools.partial(
        pl.kernel,
        out_type=jax.ShapeDtypeStruct((_N, _H), jnp.float32),
        mesh=_sc_mesh(),
        scratch_types=[
            pltpu.VMEM((2 * _TPW,), jnp.int32),
            pltpu.VMEM((2 * _CT, _H), jnp.float32),
            pltpu.VMEM((_CT, _H), jnp.float32),
            pltpu.SemaphoreType.DMA,
        ],
    )
    def combine_kernel(y_hbm, inv_hbm, out_hbm, idx_v, rows, outb, sem):
        wid = lax.axis_index("s") * 2 + lax.axis_index("c")
        tbase = wid * _TPW
        pltpu.sync_copy(inv_hbm.at[pl.ds(2 * tbase, 2 * _TPW)], idx_v)

        def chunk(c, carry):
            pltpu.async_copy(
                y_hbm.at[idx_v.at[pl.ds(c * 2 * _CT, 2 * _CT)]], rows, sem
            ).wait()

            def col(j, carry2):
                o = j * 16
                for t in range(_CT):
                    outb[t, pl.ds(o, 16)] = (
                        rows[2 * t, pl.ds(o, 16)] + rows[2 * t + 1, pl.ds(o, 16)]
                    )
                return carry2

            lax.fori_loop(0, _H // 16, col, 0)
            pltpu.sync_copy(outb, out_hbm.at[pl.ds(tbase + c * _CT, _CT)])
            return carry

        lax.fori_loop(0, _TPW // _CT, chunk, 0)

    return combine_kernel(y_pad, inv)


def _m1_body(be_ref, x_ref, wg_ref, wu_ref, h_ref):
    b = pl.program_id(1)

    @pl.when(be_ref[b] >= 0)
    def _():
        x = x_ref[...]
        wg = wg_ref[0, 0]
        wu = wu_ref[0, 0]
        g = lax.dot_general(
            x, wg, (((1,), (1,)), ((), ())), preferred_element_type=jnp.float32
        )
        u = lax.dot_general(
            x, wu, (((1,), (1,)), ((), ())), preferred_element_type=jnp.float32
        )
        h_ref[...] = g * jax.nn.sigmoid(g) * u


def _m1(x_pad, w13r, block_expert):
    grid_spec = pltpu.PrefetchScalarGridSpec(
        num_scalar_prefetch=1,
        grid=(_NN1, _NB),
        in_specs=[
            pl.BlockSpec((_BM, _H), lambda n, b, be: (b, 0)),
            pl.BlockSpec(
                (1, 1, _BN1, _H), lambda n, b, be: (jnp.maximum(be[b], 0), 0, n, 0)
            ),
            pl.BlockSpec(
                (1, 1, _BN1, _H), lambda n, b, be: (jnp.maximum(be[b], 0), 1, n, 0)
            ),
        ],
        out_specs=pl.BlockSpec((_BM, _BN1), lambda n, b, be: (b, n)),
    )
    return pl.pallas_call(
        _m1_body,
        grid_spec=grid_spec,
        out_shape=jax.ShapeDtypeStruct((_NPAD, _I), jnp.float32),
    )(block_expert, x_pad, w13r, w13r)


def _m2_body(be_ref, h_ref, w2_ref, wp_ref, y_ref):
    b = pl.program_id(1)

    @pl.when(be_ref[b] >= 0)
    def _():
        h = h_ref[...]
        w2t = w2_ref[0]
        y = lax.dot_general(
            h, w2t, (((1,), (1,)), ((), ())), preferred_element_type=jnp.float32
        )
        y_ref[...] = y * wp_ref[0, 0, :][:, None]


def _m2(h, w2_weight, wp3, block_expert):
    grid_spec = pltpu.PrefetchScalarGridSpec(
        num_scalar_prefetch=1,
        grid=(_NN2, _NB),
        in_specs=[
            pl.BlockSpec((_BM, _I), lambda n, b, be: (b, 0)),
            pl.BlockSpec(
                (1, _BN2, _I), lambda n, b, be: (jnp.maximum(be[b], 0), n, 0)
            ),
            pl.BlockSpec((1, 1, _BM), lambda n, b, be: (b, 0, 0)),
        ],
        out_specs=pl.BlockSpec((_BM, _BN2), lambda n, b, be: (b, n)),
    )
    return pl.pallas_call(
        _m2_body,
        grid_spec=grid_spec,
        out_shape=jax.ShapeDtypeStruct((_NPAD, _H), jnp.float32),
    )(block_expert, h, w2_weight, wp3)


def kernel(hidden_states, topk_ids, topk_weights, w13_weight, w2_weight):
    ids = topk_ids.reshape(-1).astype(jnp.int32)
    wts = topk_weights.reshape(-1).astype(jnp.float32)

    order = jnp.argsort(ids).astype(jnp.int32)
    sorted_ids = ids[order]
    seg = jnp.searchsorted(
        sorted_ids, jnp.arange(_E + 1, dtype=jnp.int32), side="left"
    ).astype(jnp.int32)
    padded = ((seg[1:] - seg[:-1] + _BM - 1) // _BM) * _BM
    pad_start = jnp.concatenate(
        [jnp.zeros((1,), jnp.int32), jnp.cumsum(padded).astype(jnp.int32)]
    )
    rank = jnp.arange(_NK, dtype=jnp.int32) - seg[sorted_ids]
    dest = pad_start[sorted_ids] + rank

    src_tok = jnp.zeros((_NPAD,), jnp.int32).at[dest].set(
        (order // _K).astype(jnp.int32)
    )
    w_pad = jnp.zeros((_NPAD,), jnp.float32).at[dest].set(wts[order])
    inv = jnp.zeros((_NK,), jnp.int32).at[order].set(dest)

    bstart = jnp.arange(_NB, dtype=jnp.int32) * _BM
    raw = jnp.searchsorted(pad_start, bstart, side="right").astype(jnp.int32) - 1
    block_expert = jnp.where(raw >= _E, -1, raw).astype(jnp.int32)

    x_pad = _sc_gather(hidden_states, src_tok)
    w13r = w13_weight.reshape(_E, 2, _I, _H)
    h = _m1(x_pad, w13r, block_expert)
    y_pad = _m2(h, w2_weight, w_pad.reshape(_NB, 1, _BM), block_expert)
    return _sc_combine(y_pad, inv)


# speedup vs baseline: 2.7144x; 2.7144x over previous
import functools

import jax
import jax.numpy as jnp
from jax import lax
from jax.experimental import pallas as pl
from jax.experimental.pallas import tpu as pltpu
from jax.experimental.pallas import tpu_sc as plsc

_E = 8
_K = 2
_H = 2048
_I = 1408
_N = 4096
_NK = _N * _K
_BM = 256
_NPAD = _NK + _E * _BM
_NB = _NPAD // _BM

_NW = 32
_GR = _NPAD // _NW
_GCH = 32
_TPW = _N // _NW
_CT = 16


def _sc_mesh():
    return plsc.VectorSubcoreMesh(core_axis_name="c", subcore_axis_name="s")


def _sc_gather(hidden_states, src_tok):
    @functools.partial(
        pl.kernel,
        out_type=jax.ShapeDtypeStruct((_NPAD, _H), jnp.float32),
        mesh=_sc_mesh(),
        scratch_types=[
            pltpu.VMEM((_GR,), jnp.int32),
            pltpu.VMEM((_GCH, _H), jnp.float32),
            pltpu.SemaphoreType.DMA,
        ],
    )
    def gather_kernel(x_hbm, idx_hbm, out_hbm, idx_v, buf, sem):
        wid = lax.axis_index("s") * 2 + lax.axis_index("c")
        base = wid * _GR
        pltpu.sync_copy(idx_hbm.at[pl.ds(base, _GR)], idx_v)

        def chunk(c, carry):
            cbase = c * _GCH
            pltpu.async_copy(
                x_hbm.at[idx_v.at[pl.ds(cbase, _GCH)]], buf, sem
            ).wait()
            pltpu.sync_copy(buf, out_hbm.at[pl.ds(base + cbase, _GCH)])
            return carry

        lax.fori_loop(0, _GR // _GCH, chunk, 0)

    return gather_kernel(hidden_states, src_tok)


def _sc_combine(y_pad, inv):
    @functools.partial(
        pl.kernel,
        out_type=jax.ShapeDtypeStruct((_N, _H), jnp.float32),
        mesh=_sc_mesh(),
        scratch_types=[
            pltpu.VMEM((2 * _TPW,), jnp.int32),
            pltpu.VMEM((2 * _CT, _H), jnp.float32),
            pltpu.VMEM((_CT, _H), jnp.float32),
            pltpu.SemaphoreType.DMA,
        ],
    )
    def combine_kernel(y_hbm, inv_hbm, out_hbm, idx_v, rows, outb, sem):
        wid = lax.axis_index("s") * 2 + lax.axis_index("c")
        tbase = wid * _TPW
        pltpu.sync_copy(inv_hbm.at[pl.ds(2 * tbase, 2 * _TPW)], idx_v)

        def chunk(c, carry):
            pltpu.async_copy(
                y_hbm.at[idx_v.at[pl.ds(c * 2 * _CT, 2 * _CT)]], rows, sem
            ).wait()

            def col(j, carry2):
                o = j * 16
                for t in range(_CT):
                    outb[t, pl.ds(o, 16)] = (
                        rows[2 * t, pl.ds(o, 16)] + rows[2 * t + 1, pl.ds(o, 16)]
                    )
                return carry2

            lax.fori_loop(0, _H // 16, col, 0)
            pltpu.sync_copy(outb, out_hbm.at[pl.ds(tbase + c * _CT, _CT)])
            return carry

        lax.fori_loop(0, _TPW // _CT, chunk, 0)

    return combine_kernel(y_pad, inv)


def _m1_body(be_ref, x_ref, w13_ref, h_ref):
    b = pl.program_id(0)

    @pl.when(be_ref[b] >= 0)
    def _():
        x = x_ref[...]
        wg = w13_ref[0, 0]
        wu = w13_ref[0, 1]
        g = lax.dot_general(
            x, wg, (((1,), (1,)), ((), ())), preferred_element_type=jnp.float32
        )
        u = lax.dot_general(
            x, wu, (((1,), (1,)), ((), ())), preferred_element_type=jnp.float32
        )
        h_ref[...] = g * jax.nn.sigmoid(g) * u


def _m1(x_pad, w13r, block_expert):
    grid_spec = pltpu.PrefetchScalarGridSpec(
        num_scalar_prefetch=1,
        grid=(_NB,),
        in_specs=[
            pl.BlockSpec((_BM, _H), lambda b, be: (b, 0)),
            pl.BlockSpec(
                (1, 2, _I, _H), lambda b, be: (jnp.maximum(be[b], 0), 0, 0, 0)
            ),
        ],
        out_specs=pl.BlockSpec((_BM, _I), lambda b, be: (b, 0)),
    )
    return pl.pallas_call(
        _m1_body,
        grid_spec=grid_spec,
        out_shape=jax.ShapeDtypeStruct((_NPAD, _I), jnp.float32),
    )(block_expert, x_pad, w13r)


def _m2_body(be_ref, h_ref, w2_ref, wp_ref, y_ref):
    b = pl.program_id(0)

    @pl.when(be_ref[b] >= 0)
    def _():
        h = h_ref[...]
        w2t = w2_ref[0]
        y = lax.dot_general(
            h, w2t, (((1,), (1,)), ((), ())), preferred_element_type=jnp.float32
        )
        y_ref[...] = y * wp_ref[0, 0, :][:, None]


def _m2(h, w2_weight, wp3, block_expert):
    grid_spec = pltpu.PrefetchScalarGridSpec(
        num_scalar_prefetch=1,
        grid=(_NB,),
        in_specs=[
            pl.BlockSpec((_BM, _I), lambda b, be: (b, 0)),
            pl.BlockSpec(
                (1, _H, _I), lambda b, be: (jnp.maximum(be[b], 0), 0, 0)
            ),
            pl.BlockSpec((1, 1, _BM), lambda b, be: (b, 0, 0)),
        ],
        out_specs=pl.BlockSpec((_BM, _H), lambda b, be: (b, 0)),
    )
    return pl.pallas_call(
        _m2_body,
        grid_spec=grid_spec,
        out_shape=jax.ShapeDtypeStruct((_NPAD, _H), jnp.float32),
    )(block_expert, h, w2_weight, wp3)


def kernel(hidden_states, topk_ids, topk_weights, w13_weight, w2_weight):
    ids = topk_ids.reshape(-1).astype(jnp.int32)
    wts = topk_weights.reshape(-1).astype(jnp.float32)

    order = jnp.argsort(ids).astype(jnp.int32)
    sorted_ids = ids[order]
    seg = jnp.searchsorted(
        sorted_ids, jnp.arange(_E + 1, dtype=jnp.int32), side="left"
    ).astype(jnp.int32)
    padded = ((seg[1:] - seg[:-1] + _BM - 1) // _BM) * _BM
    pad_start = jnp.concatenate(
        [jnp.zeros((1,), jnp.int32), jnp.cumsum(padded).astype(jnp.int32)]
    )
    rank = jnp.arange(_NK, dtype=jnp.int32) - seg[sorted_ids]
    dest = pad_start[sorted_ids] + rank

    src_tok = jnp.zeros((_NPAD,), jnp.int32).at[dest].set(
        (order // _K).astype(jnp.int32)
    )
    w_pad = jnp.zeros((_NPAD,), jnp.float32).at[dest].set(wts[order])
    inv = jnp.zeros((_NK,), jnp.int32).at[order].set(dest)

    bstart = jnp.arange(_NB, dtype=jnp.int32) * _BM
    raw = jnp.searchsorted(pad_start, bstart, side="right").astype(jnp.int32) - 1
    block_expert = jnp.where(raw >= _E, -1, raw).astype(jnp.int32)

    x_pad = _sc_gather(hidden_states, src_tok)
    w13r = w13_weight.reshape(_E, 2, _I, _H)
    h = _m1(x_pad, w13r, block_expert)
    y_pad = _m2(h, w2_weight, w_pad.reshape(_NB, 1, _BM), block_expert)
    return _sc_combine(y_pad, inv)


# speedup vs baseline: 2.7913x; 1.0284x over previous
import functools

import jax
import jax.numpy as jnp
from jax import lax
from jax.experimental import pallas as pl
from jax.experimental.pallas import tpu as pltpu
from jax.experimental.pallas import tpu_sc as plsc

_E = 8
_K = 2
_H = 2048
_I = 1408
_N = 4096
_NK = _N * _K
_BM = 256
_NPAD = _NK + _E * _BM
_NB = _NPAD // _BM

_NW = 32
_GR = _NPAD // _NW
_GCH = 16
_GNC = _GR // _GCH
_TPW = _N // _NW
_CT = 8
_CNC = _TPW // _CT


def _sc_mesh():
    return plsc.VectorSubcoreMesh(core_axis_name="c", subcore_axis_name="s")


def _sc_gather(hidden_states, src_tok):
    @functools.partial(
        pl.kernel,
        out_type=jax.ShapeDtypeStruct((_NPAD, _H), jnp.float32),
        mesh=_sc_mesh(),
        scratch_types=[
            pltpu.VMEM((_GR,), jnp.int32),
            pltpu.VMEM((_GCH, _H), jnp.float32),
            pltpu.VMEM((_GCH, _H), jnp.float32),
            pltpu.SemaphoreType.DMA,
            pltpu.SemaphoreType.DMA,
            pltpu.SemaphoreType.DMA,
            pltpu.SemaphoreType.DMA,
        ],
    )
    def gather_kernel(x_hbm, idx_hbm, out_hbm, idx_v, buf0, buf1, g0, g1, s0, s1):
        wid = lax.axis_index("s") * 2 + lax.axis_index("c")
        base = wid * _GR
        pltpu.sync_copy(idx_hbm.at[pl.ds(base, _GR)], idx_v)
        bufs, gsems, ssems = (buf0, buf1), (g0, g1), (s0, s1)

        def start_gather(c):
            return pltpu.async_copy(
                x_hbm.at[idx_v.at[pl.ds(c * _GCH, _GCH)]],
                bufs[c % 2],
                gsems[c % 2],
            )

        gcp = [None] * _GNC
        scp = [None] * _GNC
        gcp[0] = start_gather(0)
        for c in range(_GNC):
            if c + 1 < _GNC:
                if c >= 1:
                    scp[c - 1].wait()
                gcp[c + 1] = start_gather(c + 1)
            gcp[c].wait()
            scp[c] = pltpu.async_copy(
                bufs[c % 2],
                out_hbm.at[pl.ds(base + c * _GCH, _GCH)],
                ssems[c % 2],
            )
        scp[_GNC - 2].wait()
        scp[_GNC - 1].wait()

    return gather_kernel(hidden_states, src_tok)


def _sc_combine(y_pad, inv):
    @functools.partial(
        pl.kernel,
        out_type=jax.ShapeDtypeStruct((_N, _H), jnp.float32),
        mesh=_sc_mesh(),
        scratch_types=[
            pltpu.VMEM((2 * _TPW,), jnp.int32),
            pltpu.VMEM((2 * _CT, _H), jnp.float32),
            pltpu.VMEM((2 * _CT, _H), jnp.float32),
            pltpu.VMEM((_CT, _H), jnp.float32),
            pltpu.VMEM((_CT, _H), jnp.float32),
            pltpu.SemaphoreType.DMA,
            pltpu.SemaphoreType.DMA,
            pltpu.SemaphoreType.DMA,
            pltpu.SemaphoreType.DMA,
        ],
    )
    def combine_kernel(
        y_hbm, inv_hbm, out_hbm, idx_v, rows0, rows1, ob0, ob1, g0, g1, s0, s1
    ):
        wid = lax.axis_index("s") * 2 + lax.axis_index("c")
        tbase = wid * _TPW
        pltpu.sync_copy(inv_hbm.at[pl.ds(2 * tbase, 2 * _TPW)], idx_v)
        rows_b, ob_b, gsems, ssems = (rows0, rows1), (ob0, ob1), (g0, g1), (s0, s1)

        def start_gather(c):
            return pltpu.async_copy(
                y_hbm.at[idx_v.at[pl.ds(c * 2 * _CT, 2 * _CT)]],
                rows_b[c % 2],
                gsems[c % 2],
            )

        gcp = [None] * _CNC
        scp = [None] * _CNC
        gcp[0] = start_gather(0)
        for c in range(_CNC):
            if c + 1 < _CNC:
                gcp[c + 1] = start_gather(c + 1)
            gcp[c].wait()
            if c >= 2:
                scp[c - 2].wait()
            rows = rows_b[c % 2]
            outb = ob_b[c % 2]

            def col(j, carry2, rows=rows, outb=outb):
                o = j * 16
                for t in range(_CT):
                    outb[t, pl.ds(o, 16)] = (
                        rows[2 * t, pl.ds(o, 16)] + rows[2 * t + 1, pl.ds(o, 16)]
                    )
                return carry2

            lax.fori_loop(0, _H // 16, col, 0)
            scp[c] = pltpu.async_copy(
                outb, out_hbm.at[pl.ds(tbase + c * _CT, _CT)], ssems[c % 2]
            )
        scp[_CNC - 2].wait()
        scp[_CNC - 1].wait()

    return combine_kernel(y_pad, inv)


def _m1_body(be_ref, x_ref, w13_ref, h_ref):
    b = pl.program_id(0)

    @pl.when(be_ref[b] >= 0)
    def _():
        x = x_ref[...]
        wg = w13_ref[0, 0]
        wu = w13_ref[0, 1]
        g = lax.dot_general(
            x, wg, (((1,), (1,)), ((), ())), preferred_element_type=jnp.float32
        )
        u = lax.dot_general(
            x, wu, (((1,), (1,)), ((), ())), preferred_element_type=jnp.float32
        )
        h_ref[...] = g * jax.nn.sigmoid(g) * u


def _m1(x_pad, w13r, block_expert):
    grid_spec = pltpu.PrefetchScalarGridSpec(
        num_scalar_prefetch=1,
        grid=(_NB,),
        in_specs=[
            pl.BlockSpec((_BM, _H), lambda b, be: (b, 0)),
            pl.BlockSpec(
                (1, 2, _I, _H), lambda b, be: (jnp.maximum(be[b], 0), 0, 0, 0)
            ),
        ],
        out_specs=pl.BlockSpec((_BM, _I), lambda b, be: (b, 0)),
    )
    return pl.pallas_call(
        _m1_body,
        grid_spec=grid_spec,
        out_shape=jax.ShapeDtypeStruct((_NPAD, _I), jnp.float32),
    )(block_expert, x_pad, w13r)


def _m2_body(be_ref, h_ref, w2_ref, wp_ref, y_ref):
    b = pl.program_id(0)

    @pl.when(be_ref[b] >= 0)
    def _():
        h = h_ref[...]
        w2t = w2_ref[0]
        y = lax.dot_general(
            h, w2t, (((1,), (1,)), ((), ())), preferred_element_type=jnp.float32
        )
        y_ref[...] = y * wp_ref[0, 0, :][:, None]


def _m2(h, w2_weight, wp3, block_expert):
    grid_spec = pltpu.PrefetchScalarGridSpec(
        num_scalar_prefetch=1,
        grid=(_NB,),
        in_specs=[
            pl.BlockSpec((_BM, _I), lambda b, be: (b, 0)),
            pl.BlockSpec(
                (1, _H, _I), lambda b, be: (jnp.maximum(be[b], 0), 0, 0)
            ),
            pl.BlockSpec((1, 1, _BM), lambda b, be: (b, 0, 0)),
        ],
        out_specs=pl.BlockSpec((_BM, _H), lambda b, be: (b, 0)),
    )
    return pl.pallas_call(
        _m2_body,
        grid_spec=grid_spec,
        out_shape=jax.ShapeDtypeStruct((_NPAD, _H), jnp.float32),
    )(block_expert, h, w2_weight, wp3)


def kernel(hidden_states, topk_ids, topk_weights, w13_weight, w2_weight):
    ids = topk_ids.reshape(-1).astype(jnp.int32)
    wts = topk_weights.reshape(-1).astype(jnp.float32)

    order = jnp.argsort(ids).astype(jnp.int32)
    sorted_ids = ids[order]
    seg = jnp.searchsorted(
        sorted_ids, jnp.arange(_E + 1, dtype=jnp.int32), side="left"
    ).astype(jnp.int32)
    padded = ((seg[1:] - seg[:-1] + _BM - 1) // _BM) * _BM
    pad_start = jnp.concatenate(
        [jnp.zeros((1,), jnp.int32), jnp.cumsum(padded).astype(jnp.int32)]
    )
    rank = jnp.arange(_NK, dtype=jnp.int32) - seg[sorted_ids]
    dest = pad_start[sorted_ids] + rank

    src_tok = jnp.zeros((_NPAD,), jnp.int32).at[dest].set(
        (order // _K).astype(jnp.int32)
    )
    w_pad = jnp.zeros((_NPAD,), jnp.float32).at[dest].set(wts[order])
    inv = jnp.zeros((_NK,), jnp.int32).at[order].set(dest)

    bstart = jnp.arange(_NB, dtype=jnp.int32) * _BM
    raw = jnp.searchsorted(pad_start, bstart, side="right").astype(jnp.int32) - 1
    block_expert = jnp.where(raw >= _E, -1, raw).astype(jnp.int32)

    x_pad = _sc_gather(hidden_states, src_tok)
    w13r = w13_weight.reshape(_E, 2, _I, _H)
    h = _m1(x_pad, w13r, block_expert)
    y_pad = _m2(h, w2_weight, w_pad.reshape(_NB, 1, _BM), block_expert)
    return _sc_combine(y_pad, inv)
